# baseline (speedup 1.0000x reference)
import functools

import jax
import jax.numpy as jnp
from jax import lax
from jax.experimental import pallas as pl
from jax.experimental.pallas import tpu as pltpu
from jax.experimental.pallas import tpu_sc as plsc

E = 64
D = 768
H = 768
RH = 256
K = 2
B = 2
S = 2048
T = B * S
CAP = 256
NSLOT = E * CAP
AUX_COEF = 0.01

F32 = jnp.float32
I32 = jnp.int32


def _logits_body(h_ref, wr_ref, o_ref):
    o_ref[...] = jnp.dot(h_ref[...], wr_ref[...], preferred_element_type=F32)


def _logits_mm(hseq_flat, Wr):
    return pl.pallas_call(
        _logits_body,
        out_shape=jax.ShapeDtypeStruct((T, E), F32),
    )(hseq_flat, Wr)


def _gru_body(xw_ref, wh_ref, hs_ref):
    wh = wh_ref[...]

    def step(t, h):
        xw_t = xw_ref[:, pl.ds(t, 1), :].reshape(B, 3 * RH)
        hw = jnp.dot(h, wh, preferred_element_type=F32)
        xr = xw_t[:, :RH]
        xz = xw_t[:, RH:2 * RH]
        xn = xw_t[:, 2 * RH:]
        hr = hw[:, :RH]
        hz = hw[:, RH:2 * RH]
        hn = hw[:, 2 * RH:]
        r = jax.nn.sigmoid(xr + hr)
        z = jax.nn.sigmoid(xz + hz)
        n = jnp.tanh(xn + r * hn)
        h2 = (1.0 - z) * n + z * h
        hs_ref[:, pl.ds(t, 1), :] = h2.reshape(B, 1, RH)
        return h2

    lax.fori_loop(0, S, step, jnp.zeros((B, RH), F32))


def _gru(xw):
    return pl.pallas_call(
        _gru_body,
        in_specs=[
            pl.BlockSpec((B, S, 3 * RH), lambda: (0, 0, 0)),
            pl.BlockSpec((RH, 3 * RH), lambda: (0, 0)),
        ],
        out_specs=pl.BlockSpec((B, S, RH), lambda: (0, 0, 0)),
        out_shape=jax.ShapeDtypeStruct((B, S, RH), F32),
    )


_RB = 256
_NRB = T // _RB


def _router_body(l_ref, d1_ref, d2_ref, g1_ref, g2_ref,
                 aux_ref, counts_ref, psum_ref):
    i = pl.program_id(0)

    @pl.when(i == 0)
    def _():
        counts_ref[...] = jnp.zeros_like(counts_ref)
        psum_ref[...] = jnp.zeros_like(psum_ref)

    l = l_ref[...]

    m = jnp.max(l, axis=1, keepdims=True)
    ex = jnp.exp(l - m)
    p = ex / jnp.sum(ex, axis=1, keepdims=True)
    psum_ref[...] += jnp.sum(p, axis=0, keepdims=True)

    lane = lax.broadcasted_iota(I32, (_RB, E), 1)
    m1 = jnp.max(p, axis=1, keepdims=True)
    i1 = jnp.min(jnp.where(p == m1, lane, E), axis=1, keepdims=True)
    oh1 = (lane == i1).astype(F32)
    pm = jnp.where(lane == i1, -jnp.inf, p)
    m2 = jnp.max(pm, axis=1, keepdims=True)
    i2 = jnp.min(jnp.where(pm == m2, lane, E), axis=1, keepdims=True)
    oh2 = (lane == i2).astype(F32)

    gsum = m1 + m2
    g1 = m1 / gsum
    g2 = m2 / gsum

    row = lax.broadcasted_iota(I32, (_RB, _RB), 0)
    col = lax.broadcasted_iota(I32, (_RB, _RB), 1)
    ltri = (col < row).astype(F32)
    c = oh1 + oh2
    cumb = jnp.dot(ltri, c, preferred_element_type=F32) + counts_ref[...]
    pos1 = jnp.sum(cumb * oh1, axis=1, keepdims=True)
    pos2 = jnp.sum(cumb * oh2, axis=1, keepdims=True)

    keep1 = pos1 < CAP
    keep2 = pos2 < CAP
    d1_ref[...] = jnp.where(keep1, i1 * CAP + pos1.astype(I32), -1)
    d2_ref[...] = jnp.where(keep2, i2 * CAP + pos2.astype(I32), -1)
    g1_ref[...] = jnp.where(keep1, g1, 0.0)
    g2_ref[...] = jnp.where(keep2, g2, 0.0)

    counts_ref[...] += jnp.sum(c, axis=0, keepdims=True)

    @pl.when(i == _NRB - 1)
    def _():
        frac = counts_ref[...] / float(T)
        pmean = psum_ref[...] / float(T)
        aux_ref[...] = (AUX_COEF * E) * jnp.sum(
            frac * pmean, keepdims=True).reshape(1, 1)


def _router(logits):
    return pl.pallas_call(
        _router_body,
        grid=(_NRB,),
        in_specs=[
            pl.BlockSpec((_RB, E), lambda i: (i, 0)),
        ],
        out_specs=[
            pl.BlockSpec((_RB, 1), lambda i: (i, 0)),
            pl.BlockSpec((_RB, 1), lambda i: (i, 0)),
            pl.BlockSpec((_RB, 1), lambda i: (i, 0)),
            pl.BlockSpec((_RB, 1), lambda i: (i, 0)),
            pl.BlockSpec((1, 1), lambda i: (0, 0)),
        ],
        out_shape=[
            jax.ShapeDtypeStruct((T, 1), I32),
            jax.ShapeDtypeStruct((T, 1), I32),
            jax.ShapeDtypeStruct((T, 1), F32),
            jax.ShapeDtypeStruct((T, 1), F32),
            jax.ShapeDtypeStruct((1, 1), F32),
        ],
        scratch_shapes=[
            pltpu.VMEM((1, E), F32),
            pltpu.VMEM((1, E), F32),
        ],
        compiler_params=pltpu.CompilerParams(
            dimension_semantics=("arbitrary",)),
    )(logits)


_NC, _NS, _L = 2, 16, 16
_NW = _NC * _NS
_SLOTS_W = NSLOT // _NW
_CHUNK = 128
_ZROW = T
SRCPAD = NSLOT + 512
_FILL_W = SRCPAD // _NS
_TOK_C = T // _NS


def _invert_kernel(d1_hbm, d2_hbm, g1_hbm, g2_hbm, src_hbm, gsl_hbm,
                   fi_v, ff_v, tok_v, d1_v, d2_v, g1_v, g2_v, sem):
    wid = lax.axis_index("s")

    def fill(b, _):
        sl = pl.ds(b * _L, _L)
        fi_v[sl] = jnp.full((_L,), _ZROW, I32)
        ff_v[sl] = jnp.zeros((_L,), F32)
        return 0

    lax.fori_loop(0, _FILL_W // _L, fill, 0)
    pltpu.sync_copy(fi_v, src_hbm.at[pl.ds(wid * _FILL_W, _FILL_W)])
    pltpu.sync_copy(ff_v, gsl_hbm.at[pl.ds(wid * _FILL_W, _FILL_W)])

    plsc.subcore_barrier()

    tbase = wid * _TOK_C
    pltpu.sync_copy(d1_hbm.at[pl.ds(2 * wid, 2)], d1_v)
    pltpu.sync_copy(d2_hbm.at[pl.ds(2 * wid, 2)], d2_v)
    pltpu.sync_copy(g1_hbm.at[pl.ds(2 * wid, 2)], g1_v)
    pltpu.sync_copy(g2_hbm.at[pl.ds(2 * wid, 2)], g2_v)

    lanes = lax.iota(I32, _L)
    drop = NSLOT + wid
    for r in range(2):
        def prep(g, _):
            sl = pl.ds(g * _L, _L)
            tok_v[r, sl] = tbase + r * 128 + g * _L + lanes
            d1 = d1_v[r, sl]
            d1_v[r, sl] = jnp.where(d1 < 0, drop, d1)
            d2 = d2_v[r, sl]
            d2_v[r, sl] = jnp.where(d2 < 0, drop, d2)
            return 0

        lax.fori_loop(0, 128 // _L, prep, 0)

    for r in range(2):
        pltpu.async_copy(tok_v.at[r], src_hbm.at[d1_v.at[r]], sem).wait()
        pltpu.async_copy(tok_v.at[r], src_hbm.at[d2_v.at[r]], sem).wait()
        pltpu.async_copy(g1_v.at[r], gsl_hbm.at[d1_v.at[r]], sem).wait()
        pltpu.async_copy(g2_v.at[r], gsl_hbm.at[d2_v.at[r]], sem).wait()


def _invert(d1r, d2r, g1r, g2r):
    mesh = plsc.VectorSubcoreMesh(core_axis_name="c", subcore_axis_name="s",
                                  num_cores=1)
    return pl.kernel(
        _invert_kernel,
        mesh=mesh,
        out_type=[
            jax.ShapeDtypeStruct((SRCPAD,), I32),
            jax.ShapeDtypeStruct((SRCPAD,), F32),
        ],
        scratch_types=[
            pltpu.VMEM((_FILL_W,), I32),
            pltpu.VMEM((_FILL_W,), F32),
            pltpu.VMEM((2, 128), I32),
            pltpu.VMEM((2, 128), I32),
            pltpu.VMEM((2, 128), I32),
            pltpu.VMEM((2, 128), F32),
            pltpu.VMEM((2, 128), F32),
            pltpu.SemaphoreType.DMA,
        ],
    )(d1r, d2r, g1r, g2r)


_GC = 32
_NCH = _SLOTS_W // _GC


def _dispatch_kernel(src_hbm, xpad_hbm, buf_hbm, src_v, r0_v, r1_v, r2_v,
                     gsem, wsem0, wsem1, wsem2):
    wid = lax.axis_index("s") * _NC + lax.axis_index("c")
    base = wid * _SLOTS_W
    pltpu.sync_copy(src_hbm.at[pl.ds(base, _SLOTS_W)], src_v)

    bufs = (r0_v, r1_v, r2_v)
    wsems = (wsem0, wsem1, wsem2)

    def gather(c):
        idx = src_v.at[pl.ds(c * _GC, _GC)]
        return pltpu.async_copy(xpad_hbm.at[idx], bufs[c % 3], gsem)

    def put(c):
        return pltpu.async_copy(
            bufs[c % 3], buf_hbm.at[pl.ds(base + c * _GC, _GC)], wsems[c % 3])

    writes = [None, None, None]
    g = gather(0)
    for c in range(_NCH):
        g.wait()
        if c + 1 < _NCH:
            if writes[(c + 1) % 3] is not None:
                writes[(c + 1) % 3].wait()
            nxt = gather(c + 1)
        else:
            nxt = None
        writes[c % 3] = put(c)
        g = nxt
    for w in writes:
        if w is not None:
            w.wait()


def _dispatch(src, xpad):
    mesh = plsc.VectorSubcoreMesh(core_axis_name="c", subcore_axis_name="s")
    return pl.kernel(
        _dispatch_kernel,
        mesh=mesh,
        out_type=jax.ShapeDtypeStruct((NSLOT, D), F32),
        scratch_types=[
            pltpu.VMEM((_SLOTS_W,), I32),
            pltpu.VMEM((_GC, D), F32),
            pltpu.VMEM((_GC, D), F32),
            pltpu.VMEM((_GC, D), F32),
            pltpu.SemaphoreType.DMA,
            pltpu.SemaphoreType.DMA,
            pltpu.SemaphoreType.DMA,
            pltpu.SemaphoreType.DMA,
        ],
    )(src, xpad)


OBUF = NSLOT + CAP


def _ffn_body(buf_ref, w1_ref, w2_ref, gs_ref, out_ref):
    e = pl.program_id(0)

    @pl.when(e < E)
    def _():
        x = buf_ref[0]
        a = jnp.dot(x, w1_ref[0], preferred_element_type=F32)
        h = a * jax.nn.sigmoid(a)
        o = jnp.dot(h, w2_ref[0], preferred_element_type=F32)
        out_ref[...] = o * gs_ref[...]

    @pl.when(e == E)
    def _():
        out_ref[...] = jnp.zeros_like(out_ref)


def _ffn(buf, W1, W2, gs):
    clip = lambda e: jnp.minimum(e, E - 1)
    return pl.pallas_call(
        _ffn_body,
        grid=(E + 1,),
        in_specs=[
            pl.BlockSpec((1, CAP, D), lambda e: (clip(e), 0, 0)),
            pl.BlockSpec((1, D, H), lambda e: (clip(e), 0, 0)),
            pl.BlockSpec((1, H, D), lambda e: (clip(e), 0, 0)),
            pl.BlockSpec((CAP, 1), lambda e: (clip(e), 0)),
        ],
        out_specs=pl.BlockSpec((CAP, D), lambda e: (e, 0)),
        out_shape=jax.ShapeDtypeStruct((OBUF, D), F32),
        compiler_params=pltpu.CompilerParams(
            dimension_semantics=("arbitrary",)),
    )(buf, W1, W2, gs)


_TOK_W = T // _NW
_TCHUNK = 64


_CC = 16
_CNCH = _TOK_W // _CC


def _combine_kernel(d1_hbm, d2_hbm, ob_hbm, out_hbm,
                    d1_v, d2_v, r1a_v, r1b_v, r2a_v, r2b_v, oa_v, ob_v,
                    g1s0, g1s1, g2s0, g2s1, ws0, ws1):
    wid = lax.axis_index("s") * _NC + lax.axis_index("c")
    tbase = wid * _TOK_W

    pltpu.sync_copy(d1_hbm.at[pl.ds(tbase, _TOK_W)], d1_v)
    pltpu.sync_copy(d2_hbm.at[pl.ds(tbase, _TOK_W)], d2_v)

    def clamp(b, _):
        sl = pl.ds(b * _L, _L)
        d1 = d1_v[sl]
        d1_v[sl] = jnp.where(d1 < 0, NSLOT, d1)
        d2 = d2_v[sl]
        d2_v[sl] = jnp.where(d2 < 0, NSLOT, d2)
        return 0

    lax.fori_loop(0, _TOK_W // _L, clamp, 0)

    r1 = (r1a_v, r1b_v)
    r2 = (r2a_v, r2b_v)
    ov = (oa_v, ob_v)
    g1s = (g1s0, g1s1)
    g2s = (g2s0, g2s1)
    wss = (ws0, ws1)

    def gather(c):
        p = c % 2
        return (
            pltpu.async_copy(
                ob_hbm.at[d1_v.at[pl.ds(c * _CC, _CC)]], r1[p], g1s[p]),
            pltpu.async_copy(
                ob_hbm.at[d2_v.at[pl.ds(c * _CC, _CC)]], r2[p], g2s[p]),
        )

    for c in range(_CNCH):
        p = c % 2
        g = gather(c)
        g[0].wait()
        g[1].wait()

        def acc(i, _):
            for v in range(D // _L):
                sl = pl.ds(v * _L, _L)
                ov[p][i, sl] = r1[p][i, sl] + r2[p][i, sl]
            return 0

        lax.fori_loop(0, _CC, acc, 0)
        pltpu.async_copy(
            ov[p], out_hbm.at[pl.ds(tbase + c * _CC, _CC)], wss[p]).wait()


def _combine(d1, d2, outbuf):
    mesh = plsc.VectorSubcoreMesh(core_axis_name="c", subcore_axis_name="s")
    return pl.kernel(
        _combine_kernel,
        mesh=mesh,
        out_type=jax.ShapeDtypeStruct((T, D), F32),
        scratch_types=[
            pltpu.VMEM((_TOK_W,), I32),
            pltpu.VMEM((_TOK_W,), I32),
            pltpu.VMEM((_CC, D), F32),
            pltpu.VMEM((_CC, D), F32),
            pltpu.VMEM((_CC, D), F32),
            pltpu.VMEM((_CC, D), F32),
            pltpu.VMEM((_CC, D), F32),
            pltpu.VMEM((_CC, D), F32),
            pltpu.SemaphoreType.DMA,
            pltpu.SemaphoreType.DMA,
            pltpu.SemaphoreType.DMA,
            pltpu.SemaphoreType.DMA,
            pltpu.SemaphoreType.DMA,
            pltpu.SemaphoreType.DMA,
        ],
    )(d1, d2, outbuf)


@jax.jit
def kernel(hidden_states, Wi, Wh, Wr, W1, W2):
    flat_x = hidden_states.reshape(T, D)

    xw = jnp.einsum('bsd,dh->bsh', hidden_states, Wi)
    hs = _gru(xw)(xw, Wh)
    hseq_flat = hs.reshape(T, RH)

    logits = _logits_mm(hseq_flat, Wr)
    d1, d2, g1, g2, aux = _router(logits)
    d1 = d1.reshape(T)
    d2 = d2.reshape(T)

    src, gslot = _invert(d1.reshape(T // 128, 128), d2.reshape(T // 128, 128),
                         g1.reshape(T // 128, 128), g2.reshape(T // 128, 128))

    xpad = jnp.concatenate([flat_x, jnp.zeros((8, D), F32)], axis=0)
    buf = _dispatch(src, xpad)

    gs = gslot[:NSLOT].reshape(NSLOT, 1)
    outbuf = _ffn(buf.reshape(E, CAP, D), W1, W2, gs)
    out = _combine(d1, d2, outbuf)

    return out.reshape(B, S, D), logits, aux[0, 0]


# speedup vs baseline: 1.0102x; 1.0102x over previous
import functools

import jax
import jax.numpy as jnp
from jax import lax
from jax.experimental import pallas as pl
from jax.experimental.pallas import tpu as pltpu
from jax.experimental.pallas import tpu_sc as plsc

E = 64
D = 768
H = 768
RH = 256
K = 2
B = 2
S = 2048
T = B * S
CAP = 256
NSLOT = E * CAP
AUX_COEF = 0.01

F32 = jnp.float32
I32 = jnp.int32


def _logits_body(h_ref, wr_ref, o_ref):
    o_ref[...] = jnp.dot(h_ref[...], wr_ref[...], preferred_element_type=F32)


def _logits_mm(hseq_flat, Wr):
    return pl.pallas_call(
        _logits_body,
        out_shape=jax.ShapeDtypeStruct((T, E), F32),
    )(hseq_flat, Wr)


def _gru_body(xw_ref, wh_ref, hs_ref):
    wh = wh_ref[...]

    def step(t, h):
        xw_t = xw_ref[:, pl.ds(t, 1), :].reshape(B, 3 * RH)
        hw = jnp.dot(h, wh, preferred_element_type=F32)
        xr = xw_t[:, :RH]
        xz = xw_t[:, RH:2 * RH]
        xn = xw_t[:, 2 * RH:]
        hr = hw[:, :RH]
        hz = hw[:, RH:2 * RH]
        hn = hw[:, 2 * RH:]
        r = jax.nn.sigmoid(xr + hr)
        z = jax.nn.sigmoid(xz + hz)
        n = jnp.tanh(xn + r * hn)
        h2 = (1.0 - z) * n + z * h
        hs_ref[:, pl.ds(t, 1), :] = h2.reshape(B, 1, RH)
        return h2

    lax.fori_loop(0, S, step, jnp.zeros((B, RH), F32))


def _gru(xw):
    return pl.pallas_call(
        _gru_body,
        in_specs=[
            pl.BlockSpec((B, S, 3 * RH), lambda: (0, 0, 0)),
            pl.BlockSpec((RH, 3 * RH), lambda: (0, 0)),
        ],
        out_specs=pl.BlockSpec((B, S, RH), lambda: (0, 0, 0)),
        out_shape=jax.ShapeDtypeStruct((B, S, RH), F32),
    )


_RB = 256
_NRB = T // _RB


def _router_body(l_ref, d1_ref, d2_ref, g1_ref, g2_ref,
                 aux_ref, counts_ref, psum_ref):
    i = pl.program_id(0)

    @pl.when(i == 0)
    def _():
        counts_ref[...] = jnp.zeros_like(counts_ref)
        psum_ref[...] = jnp.zeros_like(psum_ref)

    l = l_ref[...]

    m = jnp.max(l, axis=1, keepdims=True)
    ex = jnp.exp(l - m)
    p = ex / jnp.sum(ex, axis=1, keepdims=True)
    psum_ref[...] += jnp.sum(p, axis=0, keepdims=True)

    lane = lax.broadcasted_iota(I32, (_RB, E), 1)
    m1 = jnp.max(p, axis=1, keepdims=True)
    i1 = jnp.min(jnp.where(p == m1, lane, E), axis=1, keepdims=True)
    oh1 = (lane == i1).astype(F32)
    pm = jnp.where(lane == i1, -jnp.inf, p)
    m2 = jnp.max(pm, axis=1, keepdims=True)
    i2 = jnp.min(jnp.where(pm == m2, lane, E), axis=1, keepdims=True)
    oh2 = (lane == i2).astype(F32)

    gsum = m1 + m2
    g1 = m1 / gsum
    g2 = m2 / gsum

    row = lax.broadcasted_iota(I32, (_RB, _RB), 0)
    col = lax.broadcasted_iota(I32, (_RB, _RB), 1)
    ltri = (col < row).astype(F32)
    c = oh1 + oh2
    cumb = jnp.dot(ltri, c, preferred_element_type=F32) + counts_ref[...]
    pos1 = jnp.sum(cumb * oh1, axis=1, keepdims=True)
    pos2 = jnp.sum(cumb * oh2, axis=1, keepdims=True)

    keep1 = pos1 < CAP
    keep2 = pos2 < CAP
    d1_ref[...] = jnp.where(keep1, i1 * CAP + pos1.astype(I32), -1)
    d2_ref[...] = jnp.where(keep2, i2 * CAP + pos2.astype(I32), -1)
    g1_ref[...] = jnp.where(keep1, g1, 0.0)
    g2_ref[...] = jnp.where(keep2, g2, 0.0)

    counts_ref[...] += jnp.sum(c, axis=0, keepdims=True)

    @pl.when(i == _NRB - 1)
    def _():
        frac = counts_ref[...] / float(T)
        pmean = psum_ref[...] / float(T)
        aux_ref[...] = (AUX_COEF * E) * jnp.sum(
            frac * pmean, keepdims=True).reshape(1, 1)


def _router(logits):
    return pl.pallas_call(
        _router_body,
        grid=(_NRB,),
        in_specs=[
            pl.BlockSpec((_RB, E), lambda i: (i, 0)),
        ],
        out_specs=[
            pl.BlockSpec((_RB, 1), lambda i: (i, 0)),
            pl.BlockSpec((_RB, 1), lambda i: (i, 0)),
            pl.BlockSpec((_RB, 1), lambda i: (i, 0)),
            pl.BlockSpec((_RB, 1), lambda i: (i, 0)),
            pl.BlockSpec((1, 1), lambda i: (0, 0)),
        ],
        out_shape=[
            jax.ShapeDtypeStruct((T, 1), I32),
            jax.ShapeDtypeStruct((T, 1), I32),
            jax.ShapeDtypeStruct((T, 1), F32),
            jax.ShapeDtypeStruct((T, 1), F32),
            jax.ShapeDtypeStruct((1, 1), F32),
        ],
        scratch_shapes=[
            pltpu.VMEM((1, E), F32),
            pltpu.VMEM((1, E), F32),
        ],
        compiler_params=pltpu.CompilerParams(
            dimension_semantics=("arbitrary",)),
    )(logits)


_NC, _NS, _L = 2, 16, 16
_NW = _NC * _NS
_SLOTS_W = NSLOT // _NW
_CHUNK = 128
_ZROW = T
SRCPAD = NSLOT + 512
_FILL_W = SRCPAD // _NS
_TOK_C = T // _NS


def _invert_kernel(d1_hbm, d2_hbm, g1_hbm, g2_hbm, src_hbm, gsl_hbm,
                   fi_v, ff_v, tok_v, d1_v, d2_v, g1_v, g2_v, sem):
    wid = lax.axis_index("s")

    def fill(b, _):
        sl = pl.ds(b * _L, _L)
        fi_v[sl] = jnp.full((_L,), _ZROW, I32)
        ff_v[sl] = jnp.zeros((_L,), F32)
        return 0

    lax.fori_loop(0, _FILL_W // _L, fill, 0)
    pltpu.sync_copy(fi_v, src_hbm.at[pl.ds(wid * _FILL_W, _FILL_W)])
    pltpu.sync_copy(ff_v, gsl_hbm.at[pl.ds(wid * _FILL_W, _FILL_W)])

    plsc.subcore_barrier()

    tbase = wid * _TOK_C
    pltpu.sync_copy(d1_hbm.at[pl.ds(2 * wid, 2)], d1_v)
    pltpu.sync_copy(d2_hbm.at[pl.ds(2 * wid, 2)], d2_v)
    pltpu.sync_copy(g1_hbm.at[pl.ds(2 * wid, 2)], g1_v)
    pltpu.sync_copy(g2_hbm.at[pl.ds(2 * wid, 2)], g2_v)

    lanes = lax.iota(I32, _L)
    drop = NSLOT + wid
    for r in range(2):
        def prep(g, _):
            sl = pl.ds(g * _L, _L)
            tok_v[r, sl] = tbase + r * 128 + g * _L + lanes
            d1 = d1_v[r, sl]
            d1_v[r, sl] = jnp.where(d1 < 0, drop, d1)
            d2 = d2_v[r, sl]
            d2_v[r, sl] = jnp.where(d2 < 0, drop, d2)
            return 0

        lax.fori_loop(0, 128 // _L, prep, 0)

    for r in range(2):
        pltpu.async_copy(tok_v.at[r], src_hbm.at[d1_v.at[r]], sem).wait()
        pltpu.async_copy(tok_v.at[r], src_hbm.at[d2_v.at[r]], sem).wait()
        pltpu.async_copy(g1_v.at[r], gsl_hbm.at[d1_v.at[r]], sem).wait()
        pltpu.async_copy(g2_v.at[r], gsl_hbm.at[d2_v.at[r]], sem).wait()


def _invert(d1r, d2r, g1r, g2r):
    mesh = plsc.VectorSubcoreMesh(core_axis_name="c", subcore_axis_name="s",
                                  num_cores=1)
    return pl.kernel(
        _invert_kernel,
        mesh=mesh,
        out_type=[
            jax.ShapeDtypeStruct((SRCPAD,), I32),
            jax.ShapeDtypeStruct((SRCPAD,), F32),
        ],
        scratch_types=[
            pltpu.VMEM((_FILL_W,), I32),
            pltpu.VMEM((_FILL_W,), F32),
            pltpu.VMEM((2, 128), I32),
            pltpu.VMEM((2, 128), I32),
            pltpu.VMEM((2, 128), I32),
            pltpu.VMEM((2, 128), F32),
            pltpu.VMEM((2, 128), F32),
            pltpu.SemaphoreType.DMA,
        ],
    )(d1r, d2r, g1r, g2r)


_GC = 32
_NCH = _SLOTS_W // _GC


def _dispatch_kernel(src_hbm, xpad_hbm, buf_hbm, src_v, r0_v, r1_v, r2_v,
                     gsem, wsem0, wsem1, wsem2):
    wid = lax.axis_index("s") * _NC + lax.axis_index("c")
    base = wid * _SLOTS_W
    pltpu.sync_copy(src_hbm.at[pl.ds(base, _SLOTS_W)], src_v)

    bufs = (r0_v, r1_v, r2_v)
    wsems = (wsem0, wsem1, wsem2)

    def gather(c):
        idx = src_v.at[pl.ds(c * _GC, _GC)]
        return pltpu.async_copy(xpad_hbm.at[idx], bufs[c % 3], gsem)

    def put(c):
        return pltpu.async_copy(
            bufs[c % 3], buf_hbm.at[pl.ds(base + c * _GC, _GC)], wsems[c % 3])

    writes = [None, None, None]
    g = gather(0)
    for c in range(_NCH):
        g.wait()
        if c + 1 < _NCH:
            if writes[(c + 1) % 3] is not None:
                writes[(c + 1) % 3].wait()
            nxt = gather(c + 1)
        else:
            nxt = None
        writes[c % 3] = put(c)
        g = nxt
    for w in writes:
        if w is not None:
            w.wait()


def _dispatch(src, xpad):
    mesh = plsc.VectorSubcoreMesh(core_axis_name="c", subcore_axis_name="s")
    return pl.kernel(
        _dispatch_kernel,
        mesh=mesh,
        out_type=jax.ShapeDtypeStruct((NSLOT, D), F32),
        scratch_types=[
            pltpu.VMEM((_SLOTS_W,), I32),
            pltpu.VMEM((_GC, D), F32),
            pltpu.VMEM((_GC, D), F32),
            pltpu.VMEM((_GC, D), F32),
            pltpu.SemaphoreType.DMA,
            pltpu.SemaphoreType.DMA,
            pltpu.SemaphoreType.DMA,
            pltpu.SemaphoreType.DMA,
        ],
    )(src, xpad)


OBUF = NSLOT + CAP


def _ffn_body(buf_ref, w1_ref, w2_ref, gs_ref, out_ref):
    e = pl.program_id(0)

    @pl.when(e < E)
    def _():
        x = buf_ref[0]
        a = jnp.dot(x, w1_ref[0], preferred_element_type=F32)
        h = a * jax.nn.sigmoid(a)
        o = jnp.dot(h, w2_ref[0], preferred_element_type=F32)
        out_ref[...] = o * gs_ref[...]

    @pl.when(e == E)
    def _():
        out_ref[...] = jnp.zeros_like(out_ref)


def _ffn(buf, W1, W2, gs):
    clip = lambda e: jnp.minimum(e, E - 1)
    return pl.pallas_call(
        _ffn_body,
        grid=(E + 1,),
        in_specs=[
            pl.BlockSpec((1, CAP, D), lambda e: (clip(e), 0, 0)),
            pl.BlockSpec((1, D, H), lambda e: (clip(e), 0, 0)),
            pl.BlockSpec((1, H, D), lambda e: (clip(e), 0, 0)),
            pl.BlockSpec((CAP, 1), lambda e: (clip(e), 0)),
        ],
        out_specs=pl.BlockSpec((CAP, D), lambda e: (e, 0)),
        out_shape=jax.ShapeDtypeStruct((OBUF, D), F32),
        compiler_params=pltpu.CompilerParams(
            dimension_semantics=("arbitrary",)),
    )(buf, W1, W2, gs)


_TOK_W = T // _NW
_TCHUNK = 64


_CC = 16
_CNCH = _TOK_W // _CC


def _combine_kernel(d1_hbm, d2_hbm, ob_hbm, out_hbm,
                    d1_v, d2_v, r1a_v, r1b_v, r2a_v, r2b_v, oa_v, ob_v,
                    g1s0, g1s1, g2s0, g2s1, ws0, ws1):
    wid = lax.axis_index("s") * _NC + lax.axis_index("c")
    tbase = wid * _TOK_W

    pltpu.sync_copy(d1_hbm.at[pl.ds(tbase, _TOK_W)], d1_v)
    pltpu.sync_copy(d2_hbm.at[pl.ds(tbase, _TOK_W)], d2_v)

    def clamp(b, _):
        sl = pl.ds(b * _L, _L)
        d1 = d1_v[sl]
        d1_v[sl] = jnp.where(d1 < 0, NSLOT, d1)
        d2 = d2_v[sl]
        d2_v[sl] = jnp.where(d2 < 0, NSLOT, d2)
        return 0

    lax.fori_loop(0, _TOK_W // _L, clamp, 0)

    r1 = (r1a_v, r1b_v)
    r2 = (r2a_v, r2b_v)
    ov = (oa_v, ob_v)
    g1s = (g1s0, g1s1)
    g2s = (g2s0, g2s1)
    wss = (ws0, ws1)

    def gather(c):
        p = c % 2
        return (
            pltpu.async_copy(
                ob_hbm.at[d1_v.at[pl.ds(c * _CC, _CC)]], r1[p], g1s[p]),
            pltpu.async_copy(
                ob_hbm.at[d2_v.at[pl.ds(c * _CC, _CC)]], r2[p], g2s[p]),
        )

    g = gather(0)
    writes = [None, None]
    for c in range(_CNCH):
        p = c % 2
        g[0].wait()
        g[1].wait()
        if c + 1 < _CNCH:
            g = gather(c + 1)
        if writes[p] is not None:
            writes[p].wait()

        def acc(i, _):
            for v in range(D // _L):
                sl = pl.ds(v * _L, _L)
                ov[p][i, sl] = r1[p][i, sl] + r2[p][i, sl]
            return 0

        lax.fori_loop(0, _CC, acc, 0)
        writes[p] = pltpu.async_copy(
            ov[p], out_hbm.at[pl.ds(tbase + c * _CC, _CC)], wss[p])
    writes[0].wait()
    writes[1].wait()


def _combine(d1, d2, outbuf):
    mesh = plsc.VectorSubcoreMesh(core_axis_name="c", subcore_axis_name="s")
    return pl.kernel(
        _combine_kernel,
        mesh=mesh,
        out_type=jax.ShapeDtypeStruct((T, D), F32),
        scratch_types=[
            pltpu.VMEM((_TOK_W,), I32),
            pltpu.VMEM((_TOK_W,), I32),
            pltpu.VMEM((_CC, D), F32),
            pltpu.VMEM((_CC, D), F32),
            pltpu.VMEM((_CC, D), F32),
            pltpu.VMEM((_CC, D), F32),
            pltpu.VMEM((_CC, D), F32),
            pltpu.VMEM((_CC, D), F32),
            pltpu.SemaphoreType.DMA,
            pltpu.SemaphoreType.DMA,
            pltpu.SemaphoreType.DMA,
            pltpu.SemaphoreType.DMA,
            pltpu.SemaphoreType.DMA,
            pltpu.SemaphoreType.DMA,
        ],
    )(d1, d2, outbuf)


@jax.jit
def kernel(hidden_states, Wi, Wh, Wr, W1, W2):
    flat_x = hidden_states.reshape(T, D)

    xw = jnp.einsum('bsd,dh->bsh', hidden_states, Wi)
    hs = _gru(xw)(xw, Wh)
    hseq_flat = hs.reshape(T, RH)

    logits = _logits_mm(hseq_flat, Wr)
    d1, d2, g1, g2, aux = _router(logits)
    d1 = d1.reshape(T)
    d2 = d2.reshape(T)

    src, gslot = _invert(d1.reshape(T // 128, 128), d2.reshape(T // 128, 128),
                         g1.reshape(T // 128, 128), g2.reshape(T // 128, 128))

    xpad = jnp.concatenate([flat_x, jnp.zeros((8, D), F32)], axis=0)
    buf = _dispatch(src, xpad)

    gs = gslot[:NSLOT].reshape(NSLOT, 1)
    outbuf = _ffn(buf.reshape(E, CAP, D), W1, W2, gs)
    out = _combine(d1, d2, outbuf)

    return out.reshape(B, S, D), logits, aux[0, 0]


# speedup vs baseline: 1.4867x; 1.4717x over previous
import functools

import jax
import jax.numpy as jnp
from jax import lax
from jax.experimental import pallas as pl
from jax.experimental.pallas import tpu as pltpu
from jax.experimental.pallas import tpu_sc as plsc

E = 64
D = 768
H = 768
RH = 256
K = 2
B = 2
S = 2048
T = B * S
CAP = 256
NSLOT = E * CAP
AUX_COEF = 0.01

F32 = jnp.float32
I32 = jnp.int32


def _logits_body(h_ref, wr_ref, o_ref):
    o_ref[...] = jnp.dot(h_ref[...], wr_ref[...], preferred_element_type=F32)


def _logits_mm(hseq_flat, Wr):
    return pl.pallas_call(
        _logits_body,
        out_shape=jax.ShapeDtypeStruct((T, E), F32),
    )(hseq_flat, Wr)


def _gru_body(xw_ref, wh_ref, hs_ref):
    wh = wh_ref[...]

    def step(t, h):
        xw_t = xw_ref[:, pl.ds(t, 1), :].reshape(B, 3 * RH)
        hw = jnp.dot(h, wh, preferred_element_type=F32)
        xr = xw_t[:, :RH]
        xz = xw_t[:, RH:2 * RH]
        xn = xw_t[:, 2 * RH:]
        hr = hw[:, :RH]
        hz = hw[:, RH:2 * RH]
        hn = hw[:, 2 * RH:]
        r = jax.nn.sigmoid(xr + hr)
        z = jax.nn.sigmoid(xz + hz)
        n = jnp.tanh(xn + r * hn)
        h2 = (1.0 - z) * n + z * h
        hs_ref[:, pl.ds(t, 1), :] = h2.reshape(B, 1, RH)
        return h2

    lax.fori_loop(0, S, step, jnp.zeros((B, RH), F32))


def _gru(xw):
    return pl.pallas_call(
        _gru_body,
        in_specs=[
            pl.BlockSpec((B, S, 3 * RH), lambda: (0, 0, 0)),
            pl.BlockSpec((RH, 3 * RH), lambda: (0, 0)),
        ],
        out_specs=pl.BlockSpec((B, S, RH), lambda: (0, 0, 0)),
        out_shape=jax.ShapeDtypeStruct((B, S, RH), F32),
    )


_RB = 256
_NRB = T // _RB


def _router_body(l_ref, d1_ref, d2_ref, g1_ref, g2_ref,
                 aux_ref, counts_ref, psum_ref):
    i = pl.program_id(0)

    @pl.when(i == 0)
    def _():
        counts_ref[...] = jnp.zeros_like(counts_ref)
        psum_ref[...] = jnp.zeros_like(psum_ref)

    l = l_ref[...]

    m = jnp.max(l, axis=1, keepdims=True)
    ex = jnp.exp(l - m)
    p = ex / jnp.sum(ex, axis=1, keepdims=True)
    psum_ref[...] += jnp.sum(p, axis=0, keepdims=True)

    lane = lax.broadcasted_iota(I32, (_RB, E), 1)
    m1 = jnp.max(p, axis=1, keepdims=True)
    i1 = jnp.min(jnp.where(p == m1, lane, E), axis=1, keepdims=True)
    oh1 = (lane == i1).astype(F32)
    pm = jnp.where(lane == i1, -jnp.inf, p)
    m2 = jnp.max(pm, axis=1, keepdims=True)
    i2 = jnp.min(jnp.where(pm == m2, lane, E), axis=1, keepdims=True)
    oh2 = (lane == i2).astype(F32)

    gsum = m1 + m2
    g1 = m1 / gsum
    g2 = m2 / gsum

    row = lax.broadcasted_iota(I32, (_RB, _RB), 0)
    col = lax.broadcasted_iota(I32, (_RB, _RB), 1)
    ltri = (col < row).astype(F32)
    c = oh1 + oh2
    cumb = jnp.dot(ltri, c, preferred_element_type=F32) + counts_ref[...]
    pos1 = jnp.sum(cumb * oh1, axis=1, keepdims=True)
    pos2 = jnp.sum(cumb * oh2, axis=1, keepdims=True)

    keep1 = pos1 < CAP
    keep2 = pos2 < CAP
    d1_ref[...] = jnp.where(keep1, i1 * CAP + pos1.astype(I32), -1)
    d2_ref[...] = jnp.where(keep2, i2 * CAP + pos2.astype(I32), -1)
    g1_ref[...] = jnp.where(keep1, g1, 0.0)
    g2_ref[...] = jnp.where(keep2, g2, 0.0)

    counts_ref[...] += jnp.sum(c, axis=0, keepdims=True)

    @pl.when(i == _NRB - 1)
    def _():
        frac = counts_ref[...] / float(T)
        pmean = psum_ref[...] / float(T)
        aux_ref[...] = (AUX_COEF * E) * jnp.sum(
            frac * pmean, keepdims=True).reshape(1, 1)


def _router(logits):
    return pl.pallas_call(
        _router_body,
        grid=(_NRB,),
        in_specs=[
            pl.BlockSpec((_RB, E), lambda i: (i, 0)),
        ],
        out_specs=[
            pl.BlockSpec((_RB, 1), lambda i: (i, 0)),
            pl.BlockSpec((_RB, 1), lambda i: (i, 0)),
            pl.BlockSpec((_RB, 1), lambda i: (i, 0)),
            pl.BlockSpec((_RB, 1), lambda i: (i, 0)),
            pl.BlockSpec((1, 1), lambda i: (0, 0)),
        ],
        out_shape=[
            jax.ShapeDtypeStruct((T, 1), I32),
            jax.ShapeDtypeStruct((T, 1), I32),
            jax.ShapeDtypeStruct((T, 1), F32),
            jax.ShapeDtypeStruct((T, 1), F32),
            jax.ShapeDtypeStruct((1, 1), F32),
        ],
        scratch_shapes=[
            pltpu.VMEM((1, E), F32),
            pltpu.VMEM((1, E), F32),
        ],
        compiler_params=pltpu.CompilerParams(
            dimension_semantics=("arbitrary",)),
    )(logits)


_NC, _NS, _L = 2, 16, 16
_NW = _NC * _NS
_SLOTS_W = NSLOT // _NW
_CHUNK = 128
_ZROW = T
_NZ = 512
SRCPAD = NSLOT + 512
_FILL_W = SRCPAD // _NS
_TOK_C = T // _NS


def _invert_kernel(d1_hbm, d2_hbm, g1_hbm, g2_hbm, src_hbm, gsl_hbm,
                   fi_v, ff_v, tok_v, d1_v, d2_v, g1_v, g2_v, sem):
    wid = lax.axis_index("s")

    lanes0 = lax.iota(I32, _L)
    fbase = wid * _FILL_W

    def fill(b, _):
        sl = pl.ds(b * _L, _L)
        fi_v[sl] = _ZROW + ((fbase + b * _L + lanes0) & (_NZ - 1))
        ff_v[sl] = jnp.zeros((_L,), F32)
        return 0

    lax.fori_loop(0, _FILL_W // _L, fill, 0)
    pltpu.sync_copy(fi_v, src_hbm.at[pl.ds(wid * _FILL_W, _FILL_W)])
    pltpu.sync_copy(ff_v, gsl_hbm.at[pl.ds(wid * _FILL_W, _FILL_W)])

    plsc.subcore_barrier()

    tbase = wid * _TOK_C
    pltpu.sync_copy(d1_hbm.at[pl.ds(2 * wid, 2)], d1_v)
    pltpu.sync_copy(d2_hbm.at[pl.ds(2 * wid, 2)], d2_v)
    pltpu.sync_copy(g1_hbm.at[pl.ds(2 * wid, 2)], g1_v)
    pltpu.sync_copy(g2_hbm.at[pl.ds(2 * wid, 2)], g2_v)

    lanes = lax.iota(I32, _L)
    drop = NSLOT + wid
    for r in range(2):
        def prep(g, _):
            sl = pl.ds(g * _L, _L)
            tok_v[r, sl] = tbase + r * 128 + g * _L + lanes
            d1 = d1_v[r, sl]
            d1_v[r, sl] = jnp.where(d1 < 0, drop, d1)
            d2 = d2_v[r, sl]
            d2_v[r, sl] = jnp.where(d2 < 0, drop, d2)
            return 0

        lax.fori_loop(0, 128 // _L, prep, 0)

    for r in range(2):
        pltpu.async_copy(tok_v.at[r], src_hbm.at[d1_v.at[r]], sem).wait()
        pltpu.async_copy(tok_v.at[r], src_hbm.at[d2_v.at[r]], sem).wait()
        pltpu.async_copy(g1_v.at[r], gsl_hbm.at[d1_v.at[r]], sem).wait()
        pltpu.async_copy(g2_v.at[r], gsl_hbm.at[d2_v.at[r]], sem).wait()


def _invert(d1r, d2r, g1r, g2r):
    mesh = plsc.VectorSubcoreMesh(core_axis_name="c", subcore_axis_name="s",
                                  num_cores=1)
    return pl.kernel(
        _invert_kernel,
        mesh=mesh,
        out_type=[
            jax.ShapeDtypeStruct((SRCPAD,), I32),
            jax.ShapeDtypeStruct((SRCPAD,), F32),
        ],
        scratch_types=[
            pltpu.VMEM((_FILL_W,), I32),
            pltpu.VMEM((_FILL_W,), F32),
            pltpu.VMEM((2, 128), I32),
            pltpu.VMEM((2, 128), I32),
            pltpu.VMEM((2, 128), I32),
            pltpu.VMEM((2, 128), F32),
            pltpu.VMEM((2, 128), F32),
            pltpu.SemaphoreType.DMA,
        ],
    )(d1r, d2r, g1r, g2r)


_GC = 32
_NCH = _SLOTS_W // _GC


def _dispatch_kernel(src_hbm, xpad_hbm, buf_hbm, src_v, r0_v, r1_v, r2_v,
                     gsem, wsem0, wsem1, wsem2):
    wid = lax.axis_index("s") * _NC + lax.axis_index("c")
    base = wid * _SLOTS_W
    pltpu.sync_copy(src_hbm.at[pl.ds(base, _SLOTS_W)], src_v)

    bufs = (r0_v, r1_v, r2_v)
    wsems = (wsem0, wsem1, wsem2)

    def gather(c):
        idx = src_v.at[pl.ds(c * _GC, _GC)]
        return pltpu.async_copy(xpad_hbm.at[idx], bufs[c % 3], gsem)

    def put(c):
        return pltpu.async_copy(
            bufs[c % 3], buf_hbm.at[pl.ds(base + c * _GC, _GC)], wsems[c % 3])

    writes = [None, None, None]
    g = gather(0)
    for c in range(_NCH):
        g.wait()
        if c + 1 < _NCH:
            if writes[(c + 1) % 3] is not None:
                writes[(c + 1) % 3].wait()
            nxt = gather(c + 1)
        else:
            nxt = None
        writes[c % 3] = put(c)
        g = nxt
    for w in writes:
        if w is not None:
            w.wait()


def _dispatch(src, xpad):
    mesh = plsc.VectorSubcoreMesh(core_axis_name="c", subcore_axis_name="s")
    return pl.kernel(
        _dispatch_kernel,
        mesh=mesh,
        out_type=jax.ShapeDtypeStruct((NSLOT, D), F32),
        scratch_types=[
            pltpu.VMEM((_SLOTS_W,), I32),
            pltpu.VMEM((_GC, D), F32),
            pltpu.VMEM((_GC, D), F32),
            pltpu.VMEM((_GC, D), F32),
            pltpu.SemaphoreType.DMA,
            pltpu.SemaphoreType.DMA,
            pltpu.SemaphoreType.DMA,
            pltpu.SemaphoreType.DMA,
        ],
    )(src, xpad)


OBUF = NSLOT + CAP


def _ffn_body(buf_ref, w1_ref, w2_ref, gs_ref, out_ref):
    e = pl.program_id(0)

    @pl.when(e < E)
    def _():
        x = buf_ref[0]
        a = jnp.dot(x, w1_ref[0], preferred_element_type=F32)
        h = a * jax.nn.sigmoid(a)
        o = jnp.dot(h, w2_ref[0], preferred_element_type=F32)
        out_ref[...] = o * gs_ref[...]

    @pl.when(e == E)
    def _():
        out_ref[...] = jnp.zeros_like(out_ref)


def _ffn(buf, W1, W2, gs):
    clip = lambda e: jnp.minimum(e, E - 1)
    return pl.pallas_call(
        _ffn_body,
        grid=(E + 1,),
        in_specs=[
            pl.BlockSpec((1, CAP, D), lambda e: (clip(e), 0, 0)),
            pl.BlockSpec((1, D, H), lambda e: (clip(e), 0, 0)),
            pl.BlockSpec((1, H, D), lambda e: (clip(e), 0, 0)),
            pl.BlockSpec((CAP, 1), lambda e: (clip(e), 0)),
        ],
        out_specs=pl.BlockSpec((CAP, D), lambda e: (e, 0)),
        out_shape=jax.ShapeDtypeStruct((OBUF, D), F32),
        compiler_params=pltpu.CompilerParams(
            dimension_semantics=("arbitrary",)),
    )(buf, W1, W2, gs)


_TOK_W = T // _NW
_TCHUNK = 64


_CC = 16
_CNCH = _TOK_W // _CC


def _combine_kernel(d1_hbm, d2_hbm, ob_hbm, out_hbm,
                    d1_v, d2_v, r1a_v, r1b_v, r2a_v, r2b_v, oa_v, ob_v,
                    g1s0, g1s1, g2s0, g2s1, ws0, ws1):
    wid = lax.axis_index("s") * _NC + lax.axis_index("c")
    tbase = wid * _TOK_W

    pltpu.sync_copy(d1_hbm.at[pl.ds(tbase, _TOK_W)], d1_v)
    pltpu.sync_copy(d2_hbm.at[pl.ds(tbase, _TOK_W)], d2_v)

    lanes0 = lax.iota(I32, _L)

    def clamp(b, _):
        sl = pl.ds(b * _L, _L)
        spread = NSLOT + ((b * _L + lanes0) & (CAP - 1))
        d1 = d1_v[sl]
        d1_v[sl] = jnp.where(d1 < 0, spread, d1)
        d2 = d2_v[sl]
        d2_v[sl] = jnp.where(d2 < 0, spread, d2)
        return 0

    lax.fori_loop(0, _TOK_W // _L, clamp, 0)

    r1 = (r1a_v, r1b_v)
    r2 = (r2a_v, r2b_v)
    ov = (oa_v, ob_v)
    g1s = (g1s0, g1s1)
    g2s = (g2s0, g2s1)
    wss = (ws0, ws1)

    def gather(c):
        p = c % 2
        return (
            pltpu.async_copy(
                ob_hbm.at[d1_v.at[pl.ds(c * _CC, _CC)]], r1[p], g1s[p]),
            pltpu.async_copy(
                ob_hbm.at[d2_v.at[pl.ds(c * _CC, _CC)]], r2[p], g2s[p]),
        )

    g = gather(0)
    writes = [None, None]
    for c in range(_CNCH):
        p = c % 2
        g[0].wait()
        g[1].wait()
        if c + 1 < _CNCH:
            g = gather(c + 1)
        if writes[p] is not None:
            writes[p].wait()

        def acc(i, _):
            for v in range(D // _L):
                sl = pl.ds(v * _L, _L)
                ov[p][i, sl] = r1[p][i, sl] + r2[p][i, sl]
            return 0

        lax.fori_loop(0, _CC, acc, 0)
        writes[p] = pltpu.async_copy(
            ov[p], out_hbm.at[pl.ds(tbase + c * _CC, _CC)], wss[p])
    writes[0].wait()
    writes[1].wait()


def _combine(d1, d2, outbuf):
    mesh = plsc.VectorSubcoreMesh(core_axis_name="c", subcore_axis_name="s")
    return pl.kernel(
        _combine_kernel,
        mesh=mesh,
        out_type=jax.ShapeDtypeStruct((T, D), F32),
        scratch_types=[
            pltpu.VMEM((_TOK_W,), I32),
            pltpu.VMEM((_TOK_W,), I32),
            pltpu.VMEM((_CC, D), F32),
            pltpu.VMEM((_CC, D), F32),
            pltpu.VMEM((_CC, D), F32),
            pltpu.VMEM((_CC, D), F32),
            pltpu.VMEM((_CC, D), F32),
            pltpu.VMEM((_CC, D), F32),
            pltpu.SemaphoreType.DMA,
            pltpu.SemaphoreType.DMA,
            pltpu.SemaphoreType.DMA,
            pltpu.SemaphoreType.DMA,
            pltpu.SemaphoreType.DMA,
            pltpu.SemaphoreType.DMA,
        ],
    )(d1, d2, outbuf)


@jax.jit
def kernel(hidden_states, Wi, Wh, Wr, W1, W2):
    flat_x = hidden_states.reshape(T, D)

    xw = jnp.einsum('bsd,dh->bsh', hidden_states, Wi)
    hs = _gru(xw)(xw, Wh)
    hseq_flat = hs.reshape(T, RH)

    logits = _logits_mm(hseq_flat, Wr)
    d1, d2, g1, g2, aux = _router(logits)
    d1 = d1.reshape(T)
    d2 = d2.reshape(T)

    src, gslot = _invert(d1.reshape(T // 128, 128), d2.reshape(T // 128, 128),
                         g1.reshape(T // 128, 128), g2.reshape(T // 128, 128))

    xpad = jnp.concatenate([flat_x, jnp.zeros((_NZ, D), F32)], axis=0)
    buf = _dispatch(src, xpad)

    gs = gslot[:NSLOT].reshape(NSLOT, 1)
    outbuf = _ffn(buf.reshape(E, CAP, D), W1, W2, gs)
    out = _combine(d1, d2, outbuf)

    return out.reshape(B, S, D), logits, aux[0, 0]


# speedup vs baseline: 1.6060x; 1.0802x over previous
import functools

import jax
import jax.numpy as jnp
from jax import lax
from jax.experimental import pallas as pl
from jax.experimental.pallas import tpu as pltpu
from jax.experimental.pallas import tpu_sc as plsc

E = 64
D = 768
H = 768
RH = 256
K = 2
B = 2
S = 2048
T = B * S
CAP = 256
NSLOT = E * CAP
AUX_COEF = 0.01

F32 = jnp.float32
I32 = jnp.int32


def _logits_body(h_ref, wr_ref, o_ref):
    o_ref[...] = jnp.dot(h_ref[...], wr_ref[...], preferred_element_type=F32)


def _logits_mm(hseq_flat, Wr):
    return pl.pallas_call(
        _logits_body,
        out_shape=jax.ShapeDtypeStruct((T, E), F32),
    )(hseq_flat, Wr)


def _gru_body(xw_ref, wh_ref, hs_ref):
    wh = wh_ref[...]

    def step(t, h):
        xw_t = xw_ref[:, pl.ds(t, 1), :].reshape(B, 3 * RH)
        hw = jnp.dot(h, wh, preferred_element_type=F32)
        xr = xw_t[:, :RH]
        xz = xw_t[:, RH:2 * RH]
        xn = xw_t[:, 2 * RH:]
        hr = hw[:, :RH]
        hz = hw[:, RH:2 * RH]
        hn = hw[:, 2 * RH:]
        r = jax.nn.sigmoid(xr + hr)
        z = jax.nn.sigmoid(xz + hz)
        n = jnp.tanh(xn + r * hn)
        h2 = (1.0 - z) * n + z * h
        hs_ref[:, pl.ds(t, 1), :] = h2.reshape(B, 1, RH)
        return h2

    lax.fori_loop(0, S, step, jnp.zeros((B, RH), F32), unroll=8)


def _gru(xw):
    return pl.pallas_call(
        _gru_body,
        in_specs=[
            pl.BlockSpec((B, S, 3 * RH), lambda: (0, 0, 0)),
            pl.BlockSpec((RH, 3 * RH), lambda: (0, 0)),
        ],
        out_specs=pl.BlockSpec((B, S, RH), lambda: (0, 0, 0)),
        out_shape=jax.ShapeDtypeStruct((B, S, RH), F32),
    )


_RB = 256
_NRB = T // _RB


def _router_body(l_ref, d1_ref, d2_ref, g1_ref, g2_ref,
                 aux_ref, counts_ref, psum_ref):
    i = pl.program_id(0)

    @pl.when(i == 0)
    def _():
        counts_ref[...] = jnp.zeros_like(counts_ref)
        psum_ref[...] = jnp.zeros_like(psum_ref)

    l = l_ref[...]

    m = jnp.max(l, axis=1, keepdims=True)
    ex = jnp.exp(l - m)
    p = ex / jnp.sum(ex, axis=1, keepdims=True)
    psum_ref[...] += jnp.sum(p, axis=0, keepdims=True)

    lane = lax.broadcasted_iota(I32, (_RB, E), 1)
    m1 = jnp.max(p, axis=1, keepdims=True)
    i1 = jnp.min(jnp.where(p == m1, lane, E), axis=1, keepdims=True)
    oh1 = (lane == i1).astype(F32)
    pm = jnp.where(lane == i1, -jnp.inf, p)
    m2 = jnp.max(pm, axis=1, keepdims=True)
    i2 = jnp.min(jnp.where(pm == m2, lane, E), axis=1, keepdims=True)
    oh2 = (lane == i2).astype(F32)

    gsum = m1 + m2
    g1 = m1 / gsum
    g2 = m2 / gsum

    row = lax.broadcasted_iota(I32, (_RB, _RB), 0)
    col = lax.broadcasted_iota(I32, (_RB, _RB), 1)
    ltri = (col < row).astype(F32)
    c = oh1 + oh2
    cumb = jnp.dot(ltri, c, preferred_element_type=F32) + counts_ref[...]
    pos1 = jnp.sum(cumb * oh1, axis=1, keepdims=True)
    pos2 = jnp.sum(cumb * oh2, axis=1, keepdims=True)

    keep1 = pos1 < CAP
    keep2 = pos2 < CAP
    d1_ref[...] = jnp.where(keep1, i1 * CAP + pos1.astype(I32), -1)
    d2_ref[...] = jnp.where(keep2, i2 * CAP + pos2.astype(I32), -1)
    g1_ref[...] = jnp.where(keep1, g1, 0.0)
    g2_ref[...] = jnp.where(keep2, g2, 0.0)

    counts_ref[...] += jnp.sum(c, axis=0, keepdims=True)

    @pl.when(i == _NRB - 1)
    def _():
        frac = counts_ref[...] / float(T)
        pmean = psum_ref[...] / float(T)
        aux_ref[...] = (AUX_COEF * E) * jnp.sum(
            frac * pmean, keepdims=True).reshape(1, 1)


def _router(logits):
    return pl.pallas_call(
        _router_body,
        grid=(_NRB,),
        in_specs=[
            pl.BlockSpec((_RB, E), lambda i: (i, 0)),
        ],
        out_specs=[
            pl.BlockSpec((_RB, 1), lambda i: (i, 0)),
            pl.BlockSpec((_RB, 1), lambda i: (i, 0)),
            pl.BlockSpec((_RB, 1), lambda i: (i, 0)),
            pl.BlockSpec((_RB, 1), lambda i: (i, 0)),
            pl.BlockSpec((1, 1), lambda i: (0, 0)),
        ],
        out_shape=[
            jax.ShapeDtypeStruct((T, 1), I32),
            jax.ShapeDtypeStruct((T, 1), I32),
            jax.ShapeDtypeStruct((T, 1), F32),
            jax.ShapeDtypeStruct((T, 1), F32),
            jax.ShapeDtypeStruct((1, 1), F32),
        ],
        scratch_shapes=[
            pltpu.VMEM((1, E), F32),
            pltpu.VMEM((1, E), F32),
        ],
        compiler_params=pltpu.CompilerParams(
            dimension_semantics=("arbitrary",)),
    )(logits)


_NC, _NS, _L = 2, 16, 16
_NW = _NC * _NS
_SLOTS_W = NSLOT // _NW
_CHUNK = 128
_ZROW = T
_NZ = 512
SRCPAD = NSLOT + 512
_FILL_W = SRCPAD // _NS
_TOK_C = T // _NS


def _invert_kernel(d1_hbm, d2_hbm, g1_hbm, g2_hbm, src_hbm, gsl_hbm,
                   fi_v, ff_v, tok_v, d1_v, d2_v, g1_v, g2_v, sem):
    wid = lax.axis_index("s")

    lanes0 = lax.iota(I32, _L)
    fbase = wid * _FILL_W

    def fill(b, _):
        sl = pl.ds(b * _L, _L)
        fi_v[sl] = _ZROW + ((fbase + b * _L + lanes0) & (_NZ - 1))
        ff_v[sl] = jnp.zeros((_L,), F32)
        return 0

    lax.fori_loop(0, _FILL_W // _L, fill, 0)
    pltpu.sync_copy(fi_v, src_hbm.at[pl.ds(wid * _FILL_W, _FILL_W)])
    pltpu.sync_copy(ff_v, gsl_hbm.at[pl.ds(wid * _FILL_W, _FILL_W)])

    plsc.subcore_barrier()

    tbase = wid * _TOK_C
    pltpu.sync_copy(d1_hbm.at[pl.ds(2 * wid, 2)], d1_v)
    pltpu.sync_copy(d2_hbm.at[pl.ds(2 * wid, 2)], d2_v)
    pltpu.sync_copy(g1_hbm.at[pl.ds(2 * wid, 2)], g1_v)
    pltpu.sync_copy(g2_hbm.at[pl.ds(2 * wid, 2)], g2_v)

    lanes = lax.iota(I32, _L)
    drop = NSLOT + wid
    for r in range(2):
        def prep(g, _):
            sl = pl.ds(g * _L, _L)
            tok_v[r, sl] = tbase + r * 128 + g * _L + lanes
            d1 = d1_v[r, sl]
            d1_v[r, sl] = jnp.where(d1 < 0, drop, d1)
            d2 = d2_v[r, sl]
            d2_v[r, sl] = jnp.where(d2 < 0, drop, d2)
            return 0

        lax.fori_loop(0, 128 // _L, prep, 0)

    for r in range(2):
        pltpu.async_copy(tok_v.at[r], src_hbm.at[d1_v.at[r]], sem).wait()
        pltpu.async_copy(tok_v.at[r], src_hbm.at[d2_v.at[r]], sem).wait()
        pltpu.async_copy(g1_v.at[r], gsl_hbm.at[d1_v.at[r]], sem).wait()
        pltpu.async_copy(g2_v.at[r], gsl_hbm.at[d2_v.at[r]], sem).wait()


def _invert(d1r, d2r, g1r, g2r):
    mesh = plsc.VectorSubcoreMesh(core_axis_name="c", subcore_axis_name="s",
                                  num_cores=1)
    return pl.kernel(
        _invert_kernel,
        mesh=mesh,
        out_type=[
            jax.ShapeDtypeStruct((SRCPAD,), I32),
            jax.ShapeDtypeStruct((SRCPAD,), F32),
        ],
        scratch_types=[
            pltpu.VMEM((_FILL_W,), I32),
            pltpu.VMEM((_FILL_W,), F32),
            pltpu.VMEM((2, 128), I32),
            pltpu.VMEM((2, 128), I32),
            pltpu.VMEM((2, 128), I32),
            pltpu.VMEM((2, 128), F32),
            pltpu.VMEM((2, 128), F32),
            pltpu.SemaphoreType.DMA,
        ],
    )(d1r, d2r, g1r, g2r)


_GC = 32
_NCH = _SLOTS_W // _GC


def _dispatch_kernel(src_hbm, xpad_hbm, buf_hbm, src_v, r0_v, r1_v, r2_v,
                     gsem, wsem0, wsem1, wsem2):
    wid = lax.axis_index("s") * _NC + lax.axis_index("c")
    base = wid * _SLOTS_W
    pltpu.sync_copy(src_hbm.at[pl.ds(base, _SLOTS_W)], src_v)

    bufs = (r0_v, r1_v, r2_v)
    wsems = (wsem0, wsem1, wsem2)

    def gather(c):
        idx = src_v.at[pl.ds(c * _GC, _GC)]
        return pltpu.async_copy(xpad_hbm.at[idx], bufs[c % 3], gsem)

    def put(c):
        return pltpu.async_copy(
            bufs[c % 3], buf_hbm.at[pl.ds(base + c * _GC, _GC)], wsems[c % 3])

    writes = [None, None, None]
    g = gather(0)
    for c in range(_NCH):
        g.wait()
        if c + 1 < _NCH:
            if writes[(c + 1) % 3] is not None:
                writes[(c + 1) % 3].wait()
            nxt = gather(c + 1)
        else:
            nxt = None
        writes[c % 3] = put(c)
        g = nxt
    for w in writes:
        if w is not None:
            w.wait()


def _dispatch(src, xpad):
    mesh = plsc.VectorSubcoreMesh(core_axis_name="c", subcore_axis_name="s")
    return pl.kernel(
        _dispatch_kernel,
        mesh=mesh,
        out_type=jax.ShapeDtypeStruct((NSLOT, D), F32),
        scratch_types=[
            pltpu.VMEM((_SLOTS_W,), I32),
            pltpu.VMEM((_GC, D), F32),
            pltpu.VMEM((_GC, D), F32),
            pltpu.VMEM((_GC, D), F32),
            pltpu.SemaphoreType.DMA,
            pltpu.SemaphoreType.DMA,
            pltpu.SemaphoreType.DMA,
            pltpu.SemaphoreType.DMA,
        ],
    )(src, xpad)


OBUF = NSLOT + CAP


def _ffn_body(buf_ref, w1_ref, w2_ref, gs_ref, out_ref):
    e = pl.program_id(0)

    @pl.when(e < E)
    def _():
        x = buf_ref[0].astype(jnp.bfloat16)
        w1 = w1_ref[0].astype(jnp.bfloat16)
        a = jnp.dot(x, w1, preferred_element_type=F32)
        h = (a * jax.nn.sigmoid(a)).astype(jnp.bfloat16)
        w2 = w2_ref[0].astype(jnp.bfloat16)
        o = jnp.dot(h, w2, preferred_element_type=F32)
        out_ref[...] = o * gs_ref[...]

    @pl.when(e == E)
    def _():
        out_ref[...] = jnp.zeros_like(out_ref)


def _ffn(buf, W1, W2, gs):
    clip = lambda e: jnp.minimum(e, E - 1)
    return pl.pallas_call(
        _ffn_body,
        grid=(E + 1,),
        in_specs=[
            pl.BlockSpec((1, CAP, D), lambda e: (clip(e), 0, 0)),
            pl.BlockSpec((1, D, H), lambda e: (clip(e), 0, 0)),
            pl.BlockSpec((1, H, D), lambda e: (clip(e), 0, 0)),
            pl.BlockSpec((CAP, 1), lambda e: (clip(e), 0)),
        ],
        out_specs=pl.BlockSpec((CAP, D), lambda e: (e, 0)),
        out_shape=jax.ShapeDtypeStruct((OBUF, D), F32),
        compiler_params=pltpu.CompilerParams(
            dimension_semantics=("arbitrary",)),
    )(buf, W1, W2, gs)


_TOK_W = T // _NW
_TCHUNK = 64


_CC = 16
_CNCH = _TOK_W // _CC


def _combine_kernel(d1_hbm, d2_hbm, ob_hbm, out_hbm,
                    d1_v, d2_v, r1a_v, r1b_v, r2a_v, r2b_v, oa_v, ob_v,
                    g1s0, g1s1, g2s0, g2s1, ws0, ws1):
    wid = lax.axis_index("s") * _NC + lax.axis_index("c")
    tbase = wid * _TOK_W

    pltpu.sync_copy(d1_hbm.at[pl.ds(tbase, _TOK_W)], d1_v)
    pltpu.sync_copy(d2_hbm.at[pl.ds(tbase, _TOK_W)], d2_v)

    lanes0 = lax.iota(I32, _L)

    def clamp(b, _):
        sl = pl.ds(b * _L, _L)
        spread = NSLOT + ((b * _L + lanes0) & (CAP - 1))
        d1 = d1_v[sl]
        d1_v[sl] = jnp.where(d1 < 0, spread, d1)
        d2 = d2_v[sl]
        d2_v[sl] = jnp.where(d2 < 0, spread, d2)
        return 0

    lax.fori_loop(0, _TOK_W // _L, clamp, 0)

    r1 = (r1a_v, r1b_v)
    r2 = (r2a_v, r2b_v)
    ov = (oa_v, ob_v)
    g1s = (g1s0, g1s1)
    g2s = (g2s0, g2s1)
    wss = (ws0, ws1)

    def gather(c):
        p = c % 2
        return (
            pltpu.async_copy(
                ob_hbm.at[d1_v.at[pl.ds(c * _CC, _CC)]], r1[p], g1s[p]),
            pltpu.async_copy(
                ob_hbm.at[d2_v.at[pl.ds(c * _CC, _CC)]], r2[p], g2s[p]),
        )

    g = gather(0)
    writes = [None, None]
    for c in range(_CNCH):
        p = c % 2
        g[0].wait()
        g[1].wait()
        if c + 1 < _CNCH:
            g = gather(c + 1)
        if writes[p] is not None:
            writes[p].wait()

        def acc(i, _):
            for v in range(D // _L):
                sl = pl.ds(v * _L, _L)
                ov[p][i, sl] = r1[p][i, sl] + r2[p][i, sl]
            return 0

        lax.fori_loop(0, _CC, acc, 0)
        writes[p] = pltpu.async_copy(
            ov[p], out_hbm.at[pl.ds(tbase + c * _CC, _CC)], wss[p])
    writes[0].wait()
    writes[1].wait()


def _combine(d1, d2, outbuf):
    mesh = plsc.VectorSubcoreMesh(core_axis_name="c", subcore_axis_name="s")
    return pl.kernel(
        _combine_kernel,
        mesh=mesh,
        out_type=jax.ShapeDtypeStruct((T, D), F32),
        scratch_types=[
            pltpu.VMEM((_TOK_W,), I32),
            pltpu.VMEM((_TOK_W,), I32),
            pltpu.VMEM((_CC, D), F32),
            pltpu.VMEM((_CC, D), F32),
            pltpu.VMEM((_CC, D), F32),
            pltpu.VMEM((_CC, D), F32),
            pltpu.VMEM((_CC, D), F32),
            pltpu.VMEM((_CC, D), F32),
            pltpu.SemaphoreType.DMA,
            pltpu.SemaphoreType.DMA,
            pltpu.SemaphoreType.DMA,
            pltpu.SemaphoreType.DMA,
            pltpu.SemaphoreType.DMA,
            pltpu.SemaphoreType.DMA,
        ],
    )(d1, d2, outbuf)


@jax.jit
def kernel(hidden_states, Wi, Wh, Wr, W1, W2):
    flat_x = hidden_states.reshape(T, D)

    xw = jnp.einsum('bsd,dh->bsh', hidden_states, Wi)
    hs = _gru(xw)(xw, Wh)
    hseq_flat = hs.reshape(T, RH)

    logits = _logits_mm(hseq_flat, Wr)
    d1, d2, g1, g2, aux = _router(logits)
    d1 = d1.reshape(T)
    d2 = d2.reshape(T)

    src, gslot = _invert(d1.reshape(T // 128, 128), d2.reshape(T // 128, 128),
                         g1.reshape(T // 128, 128), g2.reshape(T // 128, 128))

    xpad = jnp.concatenate([flat_x, jnp.zeros((_NZ, D), F32)], axis=0)
    buf = _dispatch(src, xpad)

    gs = gslot[:NSLOT].reshape(NSLOT, 1)
    outbuf = _ffn(buf.reshape(E, CAP, D), W1, W2, gs)
    out = _combine(d1, d2, outbuf)

    return out.reshape(B, S, D), logits, aux[0, 0]


# speedup vs baseline: 1.6216x; 1.0097x over previous
import functools

import jax
import jax.numpy as jnp
from jax import lax
from jax.experimental import pallas as pl
from jax.experimental.pallas import tpu as pltpu
from jax.experimental.pallas import tpu_sc as plsc

E = 64
D = 768
H = 768
RH = 256
K = 2
B = 2
S = 2048
T = B * S
CAP = 256
NSLOT = E * CAP
AUX_COEF = 0.01

F32 = jnp.float32
I32 = jnp.int32


def _logits_body(h_ref, wr_ref, o_ref):
    o_ref[...] = jnp.dot(h_ref[...], wr_ref[...], preferred_element_type=F32)


def _logits_mm(hseq_flat, Wr):
    return pl.pallas_call(
        _logits_body,
        out_shape=jax.ShapeDtypeStruct((T, E), F32),
    )(hseq_flat, Wr)


def _gru_body(xw_ref, wh_ref, hs_ref):
    wh = wh_ref[...]

    def step(t, h):
        xw_t = xw_ref[:, pl.ds(t, 1), :].reshape(B, 3 * RH)
        hw = jnp.dot(h, wh, preferred_element_type=F32)
        xr = xw_t[:, :RH]
        xz = xw_t[:, RH:2 * RH]
        xn = xw_t[:, 2 * RH:]
        hr = hw[:, :RH]
        hz = hw[:, RH:2 * RH]
        hn = hw[:, 2 * RH:]
        r = jax.nn.sigmoid(xr + hr)
        z = jax.nn.sigmoid(xz + hz)
        n = jnp.tanh(xn + r * hn)
        h2 = (1.0 - z) * n + z * h
        hs_ref[:, pl.ds(t, 1), :] = h2.reshape(B, 1, RH)
        return h2

    lax.fori_loop(0, S, step, jnp.zeros((B, RH), F32), unroll=16)


def _gru(xw):
    return pl.pallas_call(
        _gru_body,
        in_specs=[
            pl.BlockSpec((B, S, 3 * RH), lambda: (0, 0, 0)),
            pl.BlockSpec((RH, 3 * RH), lambda: (0, 0)),
        ],
        out_specs=pl.BlockSpec((B, S, RH), lambda: (0, 0, 0)),
        out_shape=jax.ShapeDtypeStruct((B, S, RH), F32),
    )


_RB = 256
_NRB = T // _RB


def _router_body(l_ref, d1_ref, d2_ref, g1_ref, g2_ref,
                 aux_ref, counts_ref, psum_ref):
    i = pl.program_id(0)

    @pl.when(i == 0)
    def _():
        counts_ref[...] = jnp.zeros_like(counts_ref)
        psum_ref[...] = jnp.zeros_like(psum_ref)

    l = l_ref[...]

    m = jnp.max(l, axis=1, keepdims=True)
    ex = jnp.exp(l - m)
    p = ex / jnp.sum(ex, axis=1, keepdims=True)
    psum_ref[...] += jnp.sum(p, axis=0, keepdims=True)

    lane = lax.broadcasted_iota(I32, (_RB, E), 1)
    m1 = jnp.max(p, axis=1, keepdims=True)
    i1 = jnp.min(jnp.where(p == m1, lane, E), axis=1, keepdims=True)
    oh1 = (lane == i1).astype(F32)
    pm = jnp.where(lane == i1, -jnp.inf, p)
    m2 = jnp.max(pm, axis=1, keepdims=True)
    i2 = jnp.min(jnp.where(pm == m2, lane, E), axis=1, keepdims=True)
    oh2 = (lane == i2).astype(F32)

    gsum = m1 + m2
    g1 = m1 / gsum
    g2 = m2 / gsum

    row = lax.broadcasted_iota(I32, (_RB, _RB), 0)
    col = lax.broadcasted_iota(I32, (_RB, _RB), 1)
    ltri = (col < row).astype(F32)
    c = oh1 + oh2
    cumb = jnp.dot(ltri, c, preferred_element_type=F32) + counts_ref[...]
    pos1 = jnp.sum(cumb * oh1, axis=1, keepdims=True)
    pos2 = jnp.sum(cumb * oh2, axis=1, keepdims=True)

    keep1 = pos1 < CAP
    keep2 = pos2 < CAP
    d1_ref[...] = jnp.where(keep1, i1 * CAP + pos1.astype(I32), -1)
    d2_ref[...] = jnp.where(keep2, i2 * CAP + pos2.astype(I32), -1)
    g1_ref[...] = jnp.where(keep1, g1, 0.0)
    g2_ref[...] = jnp.where(keep2, g2, 0.0)

    counts_ref[...] += jnp.sum(c, axis=0, keepdims=True)

    @pl.when(i == _NRB - 1)
    def _():
        frac = counts_ref[...] / float(T)
        pmean = psum_ref[...] / float(T)
        aux_ref[...] = (AUX_COEF * E) * jnp.sum(
            frac * pmean, keepdims=True).reshape(1, 1)


def _router(logits):
    return pl.pallas_call(
        _router_body,
        grid=(_NRB,),
        in_specs=[
            pl.BlockSpec((_RB, E), lambda i: (i, 0)),
        ],
        out_specs=[
            pl.BlockSpec((_RB, 1), lambda i: (i, 0)),
            pl.BlockSpec((_RB, 1), lambda i: (i, 0)),
            pl.BlockSpec((_RB, 1), lambda i: (i, 0)),
            pl.BlockSpec((_RB, 1), lambda i: (i, 0)),
            pl.BlockSpec((1, 1), lambda i: (0, 0)),
        ],
        out_shape=[
            jax.ShapeDtypeStruct((T, 1), I32),
            jax.ShapeDtypeStruct((T, 1), I32),
            jax.ShapeDtypeStruct((T, 1), F32),
            jax.ShapeDtypeStruct((T, 1), F32),
            jax.ShapeDtypeStruct((1, 1), F32),
        ],
        scratch_shapes=[
            pltpu.VMEM((1, E), F32),
            pltpu.VMEM((1, E), F32),
        ],
        compiler_params=pltpu.CompilerParams(
            dimension_semantics=("arbitrary",)),
    )(logits)


_NC, _NS, _L = 2, 16, 16
_NW = _NC * _NS
_SLOTS_W = NSLOT // _NW
_CHUNK = 128
_ZROW = T
_NZ = 512
SRCPAD = NSLOT + 512
_FILL_W = SRCPAD // _NS
_TOK_C = T // _NS


def _invert_kernel(d1_hbm, d2_hbm, g1_hbm, g2_hbm, src_hbm, gsl_hbm,
                   fi_v, ff_v, tok_v, d1_v, d2_v, g1_v, g2_v, sem):
    wid = lax.axis_index("s")

    lanes0 = lax.iota(I32, _L)
    fbase = wid * _FILL_W

    def fill(b, _):
        sl = pl.ds(b * _L, _L)
        fi_v[sl] = _ZROW + ((fbase + b * _L + lanes0) & (_NZ - 1))
        ff_v[sl] = jnp.zeros((_L,), F32)
        return 0

    lax.fori_loop(0, _FILL_W // _L, fill, 0)
    pltpu.sync_copy(fi_v, src_hbm.at[pl.ds(wid * _FILL_W, _FILL_W)])
    pltpu.sync_copy(ff_v, gsl_hbm.at[pl.ds(wid * _FILL_W, _FILL_W)])

    plsc.subcore_barrier()

    tbase = wid * _TOK_C
    pltpu.sync_copy(d1_hbm.at[pl.ds(2 * wid, 2)], d1_v)
    pltpu.sync_copy(d2_hbm.at[pl.ds(2 * wid, 2)], d2_v)
    pltpu.sync_copy(g1_hbm.at[pl.ds(2 * wid, 2)], g1_v)
    pltpu.sync_copy(g2_hbm.at[pl.ds(2 * wid, 2)], g2_v)

    lanes = lax.iota(I32, _L)
    drop = NSLOT + wid
    for r in range(2):
        def prep(g, _):
            sl = pl.ds(g * _L, _L)
            tok_v[r, sl] = tbase + r * 128 + g * _L + lanes
            d1 = d1_v[r, sl]
            d1_v[r, sl] = jnp.where(d1 < 0, drop, d1)
            d2 = d2_v[r, sl]
            d2_v[r, sl] = jnp.where(d2 < 0, drop, d2)
            return 0

        lax.fori_loop(0, 128 // _L, prep, 0)

    for r in range(2):
        pltpu.async_copy(tok_v.at[r], src_hbm.at[d1_v.at[r]], sem).wait()
        pltpu.async_copy(tok_v.at[r], src_hbm.at[d2_v.at[r]], sem).wait()
        pltpu.async_copy(g1_v.at[r], gsl_hbm.at[d1_v.at[r]], sem).wait()
        pltpu.async_copy(g2_v.at[r], gsl_hbm.at[d2_v.at[r]], sem).wait()


def _invert(d1r, d2r, g1r, g2r):
    mesh = plsc.VectorSubcoreMesh(core_axis_name="c", subcore_axis_name="s",
                                  num_cores=1)
    return pl.kernel(
        _invert_kernel,
        mesh=mesh,
        out_type=[
            jax.ShapeDtypeStruct((SRCPAD,), I32),
            jax.ShapeDtypeStruct((SRCPAD,), F32),
        ],
        scratch_types=[
            pltpu.VMEM((_FILL_W,), I32),
            pltpu.VMEM((_FILL_W,), F32),
            pltpu.VMEM((2, 128), I32),
            pltpu.VMEM((2, 128), I32),
            pltpu.VMEM((2, 128), I32),
            pltpu.VMEM((2, 128), F32),
            pltpu.VMEM((2, 128), F32),
            pltpu.SemaphoreType.DMA,
        ],
    )(d1r, d2r, g1r, g2r)


_GC = 32
_NCH = _SLOTS_W // _GC


def _dispatch_kernel(src_hbm, xpad_hbm, buf_hbm, src_v, r0_v, r1_v, r2_v,
                     gsem, wsem0, wsem1, wsem2):
    wid = lax.axis_index("s") * _NC + lax.axis_index("c")
    base = wid * _SLOTS_W
    pltpu.sync_copy(src_hbm.at[pl.ds(base, _SLOTS_W)], src_v)

    bufs = (r0_v, r1_v, r2_v)
    wsems = (wsem0, wsem1, wsem2)

    def gather(c):
        idx = src_v.at[pl.ds(c * _GC, _GC)]
        return pltpu.async_copy(xpad_hbm.at[idx], bufs[c % 3], gsem)

    def put(c):
        return pltpu.async_copy(
            bufs[c % 3], buf_hbm.at[pl.ds(base + c * _GC, _GC)], wsems[c % 3])

    writes = [None, None, None]
    g = gather(0)
    for c in range(_NCH):
        g.wait()
        if c + 1 < _NCH:
            if writes[(c + 1) % 3] is not None:
                writes[(c + 1) % 3].wait()
            nxt = gather(c + 1)
        else:
            nxt = None
        writes[c % 3] = put(c)
        g = nxt
    for w in writes:
        if w is not None:
            w.wait()


def _dispatch(src, xpad):
    mesh = plsc.VectorSubcoreMesh(core_axis_name="c", subcore_axis_name="s")
    return pl.kernel(
        _dispatch_kernel,
        mesh=mesh,
        out_type=jax.ShapeDtypeStruct((NSLOT, D), F32),
        scratch_types=[
            pltpu.VMEM((_SLOTS_W,), I32),
            pltpu.VMEM((_GC, D), F32),
            pltpu.VMEM((_GC, D), F32),
            pltpu.VMEM((_GC, D), F32),
            pltpu.SemaphoreType.DMA,
            pltpu.SemaphoreType.DMA,
            pltpu.SemaphoreType.DMA,
            pltpu.SemaphoreType.DMA,
        ],
    )(src, xpad)


OBUF = NSLOT + CAP


def _ffn_body(buf_ref, w1_ref, w2_ref, gs_ref, out_ref):
    e = pl.program_id(0)

    @pl.when(e < E)
    def _():
        x = buf_ref[0].astype(jnp.bfloat16)
        w1 = w1_ref[0].astype(jnp.bfloat16)
        a = jnp.dot(x, w1, preferred_element_type=F32)
        h = (a * jax.nn.sigmoid(a)).astype(jnp.bfloat16)
        w2 = w2_ref[0].astype(jnp.bfloat16)
        o = jnp.dot(h, w2, preferred_element_type=F32)
        out_ref[...] = o * gs_ref[...]

    @pl.when(e == E)
    def _():
        out_ref[...] = jnp.zeros_like(out_ref)


def _ffn(buf, W1, W2, gs):
    clip = lambda e: jnp.minimum(e, E - 1)
    return pl.pallas_call(
        _ffn_body,
        grid=(E + 1,),
        in_specs=[
            pl.BlockSpec((1, CAP, D), lambda e: (clip(e), 0, 0)),
            pl.BlockSpec((1, D, H), lambda e: (clip(e), 0, 0)),
            pl.BlockSpec((1, H, D), lambda e: (clip(e), 0, 0)),
            pl.BlockSpec((CAP, 1), lambda e: (clip(e), 0)),
        ],
        out_specs=pl.BlockSpec((CAP, D), lambda e: (e, 0)),
        out_shape=jax.ShapeDtypeStruct((OBUF, D), F32),
        compiler_params=pltpu.CompilerParams(
            dimension_semantics=("arbitrary",)),
    )(buf, W1, W2, gs)


_TOK_W = T // _NW
_TCHUNK = 64


_CC = 16
_CNCH = _TOK_W // _CC


def _combine_kernel(d1_hbm, d2_hbm, ob_hbm, out_hbm,
                    d1_v, d2_v, r1a_v, r1b_v, r2a_v, r2b_v, oa_v, ob_v,
                    g1s0, g1s1, g2s0, g2s1, ws0, ws1):
    wid = lax.axis_index("s") * _NC + lax.axis_index("c")
    tbase = wid * _TOK_W

    pltpu.sync_copy(d1_hbm.at[pl.ds(tbase, _TOK_W)], d1_v)
    pltpu.sync_copy(d2_hbm.at[pl.ds(tbase, _TOK_W)], d2_v)

    lanes0 = lax.iota(I32, _L)

    def clamp(b, _):
        sl = pl.ds(b * _L, _L)
        spread = NSLOT + ((b * _L + lanes0) & (CAP - 1))
        d1 = d1_v[sl]
        d1_v[sl] = jnp.where(d1 < 0, spread, d1)
        d2 = d2_v[sl]
        d2_v[sl] = jnp.where(d2 < 0, spread, d2)
        return 0

    lax.fori_loop(0, _TOK_W // _L, clamp, 0)

    r1 = (r1a_v, r1b_v)
    r2 = (r2a_v, r2b_v)
    ov = (oa_v, ob_v)
    g1s = (g1s0, g1s1)
    g2s = (g2s0, g2s1)
    wss = (ws0, ws1)

    def gather(c):
        p = c % 2
        return (
            pltpu.async_copy(
                ob_hbm.at[d1_v.at[pl.ds(c * _CC, _CC)]], r1[p], g1s[p]),
            pltpu.async_copy(
                ob_hbm.at[d2_v.at[pl.ds(c * _CC, _CC)]], r2[p], g2s[p]),
        )

    g = gather(0)
    writes = [None, None]
    for c in range(_CNCH):
        p = c % 2
        g[0].wait()
        g[1].wait()
        if c + 1 < _CNCH:
            g = gather(c + 1)
        if writes[p] is not None:
            writes[p].wait()

        def acc(i, _):
            for v in range(D // _L):
                sl = pl.ds(v * _L, _L)
                ov[p][i, sl] = r1[p][i, sl] + r2[p][i, sl]
            return 0

        lax.fori_loop(0, _CC, acc, 0)
        writes[p] = pltpu.async_copy(
            ov[p], out_hbm.at[pl.ds(tbase + c * _CC, _CC)], wss[p])
    writes[0].wait()
    writes[1].wait()


def _combine(d1, d2, outbuf):
    mesh = plsc.VectorSubcoreMesh(core_axis_name="c", subcore_axis_name="s")
    return pl.kernel(
        _combine_kernel,
        mesh=mesh,
        out_type=jax.ShapeDtypeStruct((T, D), F32),
        scratch_types=[
            pltpu.VMEM((_TOK_W,), I32),
            pltpu.VMEM((_TOK_W,), I32),
            pltpu.VMEM((_CC, D), F32),
            pltpu.VMEM((_CC, D), F32),
            pltpu.VMEM((_CC, D), F32),
            pltpu.VMEM((_CC, D), F32),
            pltpu.VMEM((_CC, D), F32),
            pltpu.VMEM((_CC, D), F32),
            pltpu.SemaphoreType.DMA,
            pltpu.SemaphoreType.DMA,
            pltpu.SemaphoreType.DMA,
            pltpu.SemaphoreType.DMA,
            pltpu.SemaphoreType.DMA,
            pltpu.SemaphoreType.DMA,
        ],
    )(d1, d2, outbuf)


@jax.jit
def kernel(hidden_states, Wi, Wh, Wr, W1, W2):
    flat_x = hidden_states.reshape(T, D)

    xw = jnp.einsum('bsd,dh->bsh', hidden_states, Wi)
    hs = _gru(xw)(xw, Wh)
    hseq_flat = hs.reshape(T, RH)

    logits = _logits_mm(hseq_flat, Wr)
    d1, d2, g1, g2, aux = _router(logits)
    d1 = d1.reshape(T)
    d2 = d2.reshape(T)

    src, gslot = _invert(d1.reshape(T // 128, 128), d2.reshape(T // 128, 128),
                         g1.reshape(T // 128, 128), g2.reshape(T // 128, 128))

    xpad = jnp.concatenate([flat_x, jnp.zeros((_NZ, D), F32)], axis=0)
    buf = _dispatch(src, xpad)

    gs = gslot[:NSLOT].reshape(NSLOT, 1)
    outbuf = _ffn(buf.reshape(E, CAP, D), W1, W2, gs)
    out = _combine(d1, d2, outbuf)

    return out.reshape(B, S, D), logits, aux[0, 0]
